# Initial kernel scaffold; baseline (speedup 1.0000x reference)
import jax
import jax.numpy as jnp
from jax.experimental import pallas as pl


def kernel(feat, edge_index, edge_type, node_graph_ids, W_rel1, W_self1, b1, W_rel2, W_self2, b2, Wc1, bc1, Wc2, bc2):
    raise NotImplementedError("write your pallas kernel here")


# speedup vs baseline: 1.1402x; 1.1402x over previous
import functools

import jax
import jax.numpy as jnp
from jax import lax
from jax.experimental import pallas as pl
from jax.experimental.pallas import tpu as pltpu
from jax.experimental.pallas import tpu_sc as plsc

N = 10000
E = 320000
R = 34
D = 128
G = 64

NC = 2
NS = 16
NW = NC * NS

CH = 128
NCHUNK = 80
EP = NW * NCHUNK * CH
N8 = 10240
ROWS_PER_TILE = N8 // NS

BN = 400
NB = N // BN


def _relmm_body(h_ref, w_ref, out_ref):
    out_ref[...] = jnp.dot(h_ref[...], w_ref[0],
                           preferred_element_type=jnp.float32)


def _rel_matmul(h, W_rel):
    return pl.pallas_call(
        _relmm_body,
        grid=(NB, R),
        in_specs=[
            pl.BlockSpec((BN, D), lambda i, r: (i, 0)),
            pl.BlockSpec((1, D, D), lambda i, r: (r, 0, 0)),
        ],
        out_specs=pl.BlockSpec((BN, D), lambda i, r: (r * NB + i, 0)),
        out_shape=jax.ShapeDtypeStruct((R * N, D), jnp.float32),
    )(h, W_rel)


def _gidx_body(src_ref, ety_ref, out_ref):
    out_ref[...] = ety_ref[...] * N + src_ref[...]


def _gidx_compute(srcF, etyF):
    return pl.pallas_call(
        _gidx_body,
        grid=(EP // 8192,),
        in_specs=[
            pl.BlockSpec((8, 1024), lambda i: (i, 0)),
            pl.BlockSpec((8, 1024), lambda i: (i, 0)),
        ],
        out_specs=pl.BlockSpec((8, 1024), lambda i: (i, 0)),
        out_shape=jax.ShapeDtypeStruct((EP // 1024, 1024), jnp.int32),
    )(srcF.reshape(EP // 1024, 1024), etyF.reshape(EP // 1024, 1024))


GRP = 16


def _sc_deg_body(dst_hbm, deg_hbm, dv, zbd, ones, degS):
    cid = lax.axis_index("c")
    sid = lax.axis_index("s")
    wid = sid * NC + cid
    base = sid * ROWS_PER_TILE

    zero16 = jnp.zeros((16,), jnp.float32)

    def _zbd(i, _):
        zbd[i // 8, pl.ds((i % 8) * 16, 16)] = zero16
        return 0
    lax.fori_loop(0, 128, _zbd, 0)

    def _ones(i, _):
        ones[i // 8, pl.ds((i % 8) * 16, 16)] = jnp.ones((16,), jnp.float32)
        return 0
    lax.fori_loop(0, CH * 8, _ones, 0)

    def _zd(k, _):
        pltpu.sync_copy(zbd, degS.at[pl.ds(base + k * 16, 16)])
        return 0
    lax.fori_loop(0, ROWS_PER_TILE // 16, _zd, 0)

    plsc.subcore_barrier()

    def _group(g, _):
        pltpu.sync_copy(dst_hbm.at[wid, pl.ds(g * GRP, GRP)], dv)

        def _edge(j, _):
            pltpu.sync_copy(ones, degS.at[dv.at[j]], add=True)
            return 0
        lax.fori_loop(0, GRP, _edge, 0)
        return 0
    lax.fori_loop(0, NCHUNK // GRP, _group, 0)

    plsc.subcore_barrier()

    pltpu.sync_copy(degS.at[pl.ds(base, ROWS_PER_TILE)],
                    deg_hbm.at[cid, pl.ds(base, ROWS_PER_TILE)])


def _sc_deg_pass(dstR):
    mesh = plsc.VectorSubcoreMesh(core_axis_name="c", subcore_axis_name="s")
    k = pl.kernel(
        _sc_deg_body,
        out_type=jax.ShapeDtypeStruct((NC, N8, D), jnp.float32),
        mesh=mesh,
        scratch_types=[
            pltpu.VMEM((GRP, CH), jnp.int32),
            pltpu.VMEM((16, D), jnp.float32),
            pltpu.VMEM((CH, D), jnp.float32),
            pltpu.VMEM_SHARED((N8, D), jnp.float32),
        ],
    )
    return k(dstR)


def _sc_edge_body(*refs):
    (hw_hbm, gix_hbm, dst_hbm, agg_hbm,
     gv, dv, rows, zb, aggS, sem) = refs

    cid = lax.axis_index("c")
    sid = lax.axis_index("s")
    wid = sid * NC + cid
    base = sid * ROWS_PER_TILE

    zero16 = jnp.zeros((16,), jnp.float32)

    def _zb(i, _):
        zb[i // 8, pl.ds((i % 8) * 16, 16)] = zero16
        return 0
    lax.fori_loop(0, 128, _zb, 0)

    def _za(k, _):
        pltpu.sync_copy(zb, aggS.at[pl.ds(base + k * 16, 16)])
        return 0
    lax.fori_loop(0, ROWS_PER_TILE // 16, _za, 0)

    plsc.subcore_barrier()

    def _group(g, _):
        pltpu.sync_copy(gix_hbm.at[wid, pl.ds(g * GRP, GRP)], gv)
        pltpu.sync_copy(dst_hbm.at[wid, pl.ds(g * GRP, GRP)], dv)

        def _edge(j, _):
            pltpu.async_copy(hw_hbm.at[gv.at[j]], rows, sem).wait()
            pltpu.sync_copy(rows, aggS.at[dv.at[j]], add=True)
            return 0
        lax.fori_loop(0, GRP, _edge, 0)
        return 0
    lax.fori_loop(0, NCHUNK // GRP, _group, 0)

    plsc.subcore_barrier()

    pltpu.sync_copy(aggS.at[pl.ds(base, ROWS_PER_TILE)],
                    agg_hbm.at[cid, pl.ds(base, ROWS_PER_TILE)])


def _sc_edge_pass(hW, gixR, dstR):
    mesh = plsc.VectorSubcoreMesh(core_axis_name="c", subcore_axis_name="s")
    k = pl.kernel(
        _sc_edge_body,
        out_type=jax.ShapeDtypeStruct((NC, N8, D), jnp.float32),
        mesh=mesh,
        scratch_types=[
            pltpu.VMEM((GRP, CH), jnp.int32),
            pltpu.VMEM((GRP, CH), jnp.int32),
            pltpu.VMEM((CH, D), jnp.float32),
            pltpu.VMEM((16, D), jnp.float32),
            pltpu.VMEM_SHARED((N8, D), jnp.float32),
            pltpu.SemaphoreType.DMA,
        ],
    )
    return k(hW, gixR, dstR)


def _combine_body(agg_ref, deg_ref, h_ref, w_ref, b_ref, out_ref):
    a = agg_ref[0] + agg_ref[1]
    d = deg_ref[0, :, 0] + deg_ref[1, :, 0]
    inv = 1.0 / jnp.maximum(d, 1.0)
    m = a * inv[:, None]
    s = jnp.dot(h_ref[...], w_ref[...], preferred_element_type=jnp.float32)
    out_ref[...] = jnp.maximum(m + s + b_ref[...], 0.0)


def _combine(agg, deg, h, W_self, b):
    return pl.pallas_call(
        _combine_body,
        grid=(NB,),
        in_specs=[
            pl.BlockSpec((NC, BN, D), lambda i: (0, i, 0)),
            pl.BlockSpec((NC, BN, D), lambda i: (0, i, 0)),
            pl.BlockSpec((BN, D), lambda i: (i, 0)),
            pl.BlockSpec((D, D), lambda i: (0, 0)),
            pl.BlockSpec((1, D), lambda i: (0, 0)),
        ],
        out_specs=pl.BlockSpec((BN, D), lambda i: (i, 0)),
        out_shape=jax.ShapeDtypeStruct((N, D), jnp.float32),
    )(agg, deg, h, W_self, b)


def _readout_body(agg_ref, deg_ref, h_ref, w_ref, b_ref, ids_ref,
                  wc1_ref, bc1_ref, wc2_ref, bc2_ref, out_ref,
                  pooled_s, cnt_s):
    i = pl.program_id(0)

    @pl.when(i == 0)
    def _init():
        pooled_s[...] = jnp.zeros((G, D), jnp.float32)
        cnt_s[...] = jnp.zeros((G, 128), jnp.float32)

    a = agg_ref[0] + agg_ref[1]
    d = deg_ref[0, :, 0] + deg_ref[1, :, 0]
    inv = 1.0 / jnp.maximum(d, 1.0)
    m = a * inv[:, None]
    s = jnp.dot(h_ref[...], w_ref[...], preferred_element_type=jnp.float32)
    h2 = jnp.maximum(m + s + b_ref[...], 0.0)

    ids = ids_ref[0, 0, :]
    gi = lax.broadcasted_iota(jnp.int32, (G, BN), 0)
    mask = (ids[None, :] == gi).astype(jnp.float32)
    pooled_s[...] += jnp.dot(mask, h2, preferred_element_type=jnp.float32,
                             precision=lax.Precision.HIGHEST)
    cnt_s[:, 0:1] += jnp.sum(mask, axis=1, keepdims=True)

    @pl.when(i == NB - 1)
    def _final():
        cnt = jnp.maximum(cnt_s[:, 0:1], 1.0)
        pooled = pooled_s[...] / cnt
        hid = jnp.maximum(
            jnp.dot(pooled, wc1_ref[...], preferred_element_type=jnp.float32)
            + bc1_ref[...], 0.0)
        logits = jnp.sum(hid * wc2_ref[...], axis=1, keepdims=True) + bc2_ref[...]
        out_ref[...] = logits


def _readout(agg, deg, h, W_self, b, ids3, Wc1, bc1, wc2row, bc2):
    return pl.pallas_call(
        _readout_body,
        grid=(NB,),
        in_specs=[
            pl.BlockSpec((NC, BN, D), lambda i: (0, i, 0)),
            pl.BlockSpec((NC, BN, D), lambda i: (0, i, 0)),
            pl.BlockSpec((BN, D), lambda i: (i, 0)),
            pl.BlockSpec((D, D), lambda i: (0, 0)),
            pl.BlockSpec((1, D), lambda i: (0, 0)),
            pl.BlockSpec((1, 1, BN), lambda i: (i, 0, 0)),
            pl.BlockSpec((D, D), lambda i: (0, 0)),
            pl.BlockSpec((1, D), lambda i: (0, 0)),
            pl.BlockSpec((1, D), lambda i: (0, 0)),
            pl.BlockSpec((1, 1), lambda i: (0, 0)),
        ],
        out_specs=pl.BlockSpec((G, 1), lambda i: (0, 0)),
        out_shape=jax.ShapeDtypeStruct((G, 1), jnp.float32),
        scratch_shapes=[
            pltpu.VMEM((G, D), jnp.float32),
            pltpu.VMEM((G, 128), jnp.float32),
        ],
    )(agg, deg, h, W_self, b, ids3, Wc1, bc1, wc2row, bc2)


def kernel(feat, edge_index, edge_type, node_graph_ids,
           W_rel1, W_self1, b1, W_rel2, W_self2, b2,
           Wc1, bc1, Wc2, bc2):
    src = edge_index[0].astype(jnp.int32)
    dst = edge_index[1].astype(jnp.int32)
    ety = edge_type.astype(jnp.int32)

    pad = EP - E
    srcF = jnp.concatenate([src, jnp.zeros((pad,), jnp.int32)])
    etyF = jnp.concatenate([ety, jnp.zeros((pad,), jnp.int32)])
    dstR = jnp.concatenate([dst, jnp.full((pad,), N, jnp.int32)]
                           ).reshape(NW, NCHUNK, CH)
    gixR = _gidx_compute(srcF, etyF).reshape(NW, NCHUNK, CH)
    ids3 = node_graph_ids.astype(jnp.int32).reshape(NB, 1, BN)

    b1r = b1.reshape(1, D)
    b2r = b2.reshape(1, D)
    bc1r = bc1.reshape(1, D)
    wc2row = Wc2.reshape(1, D)
    bc2r = bc2.reshape(1, 1)

    deg = _sc_deg_pass(dstR)
    hW1 = _rel_matmul(feat, W_rel1)
    agg1 = _sc_edge_pass(hW1, gixR, dstR)
    h1 = _combine(agg1, deg, feat, W_self1, b1r)

    hW2 = _rel_matmul(h1, W_rel2)
    agg2 = _sc_edge_pass(hW2, gixR, dstR)
    logits = _readout(agg2, deg, h1, W_self2, b2r, ids3, Wc1, bc1r,
                      wc2row, bc2r)
    return logits


# speedup vs baseline: 2.0579x; 1.8049x over previous
import functools

import jax
import jax.numpy as jnp
from jax import lax
from jax.experimental import pallas as pl
from jax.experimental.pallas import tpu as pltpu
from jax.experimental.pallas import tpu_sc as plsc

N = 10000
E = 320000
R = 34
D = 128
G = 64

NC = 2
NS = 16
NW = NC * NS

CH = 128
NCHUNK = 80
EP = NW * NCHUNK * CH
N8 = 10240
ROWS_PER_TILE = N8 // NS

BN = 400
NB = N // BN


def _relmm_body(h_ref, w_ref, out_ref):
    h = h_ref[...]
    for r in range(R):
        out_ref[r] = jnp.dot(h, w_ref[r], preferred_element_type=jnp.float32)


def _rel_matmul(h, W_rel):
    out = pl.pallas_call(
        _relmm_body,
        grid=(NB,),
        in_specs=[
            pl.BlockSpec((BN, D), lambda i: (i, 0)),
            pl.BlockSpec((R, D, D), lambda i: (0, 0, 0)),
        ],
        out_specs=pl.BlockSpec((R, BN, D), lambda i: (0, i, 0)),
        out_shape=jax.ShapeDtypeStruct((R, N, D), jnp.float32),
    )(h, W_rel)
    return out.reshape(R * N, D)


def _gidx_body(src_ref, ety_ref, out_ref):
    out_ref[...] = ety_ref[...] * N + src_ref[...]


def _gidx_compute(srcF, etyF):
    return pl.pallas_call(
        _gidx_body,
        grid=(EP // 8192,),
        in_specs=[
            pl.BlockSpec((8, 1024), lambda i: (i, 0)),
            pl.BlockSpec((8, 1024), lambda i: (i, 0)),
        ],
        out_specs=pl.BlockSpec((8, 1024), lambda i: (i, 0)),
        out_shape=jax.ShapeDtypeStruct((EP // 1024, 1024), jnp.int32),
    )(srcF.reshape(EP // 1024, 1024), etyF.reshape(EP // 1024, 1024))


GRP = 16


def _sc_deg_body(dst_hbm, deg_hbm, dv, zbd, ones, degS):
    cid = lax.axis_index("c")
    sid = lax.axis_index("s")
    wid = sid * NC + cid
    base = sid * ROWS_PER_TILE

    zero16 = jnp.zeros((16,), jnp.float32)

    def _zbd(i, _):
        zbd[i // 8, pl.ds((i % 8) * 16, 16)] = zero16
        return 0
    lax.fori_loop(0, 128, _zbd, 0)

    def _ones(i, _):
        ones[i // 8, pl.ds((i % 8) * 16, 16)] = jnp.ones((16,), jnp.float32)
        return 0
    lax.fori_loop(0, CH * 8, _ones, 0)

    def _zd(k, _):
        pltpu.sync_copy(zbd, degS.at[pl.ds(base + k * 16, 16)])
        return 0
    lax.fori_loop(0, ROWS_PER_TILE // 16, _zd, 0)

    plsc.subcore_barrier()

    def _group(g, _):
        pltpu.sync_copy(dst_hbm.at[wid, pl.ds(g * GRP, GRP)], dv)

        def _edge(j, _):
            pltpu.sync_copy(ones, degS.at[dv.at[j]], add=True)
            return 0
        lax.fori_loop(0, GRP, _edge, 0)
        return 0
    lax.fori_loop(0, NCHUNK // GRP, _group, 0)

    plsc.subcore_barrier()

    pltpu.sync_copy(degS.at[pl.ds(base, ROWS_PER_TILE)],
                    deg_hbm.at[cid, pl.ds(base, ROWS_PER_TILE)])


def _sc_deg_pass(dstR):
    mesh = plsc.VectorSubcoreMesh(core_axis_name="c", subcore_axis_name="s")
    k = pl.kernel(
        _sc_deg_body,
        out_type=jax.ShapeDtypeStruct((NC, N8, D), jnp.float32),
        mesh=mesh,
        scratch_types=[
            pltpu.VMEM((GRP, CH), jnp.int32),
            pltpu.VMEM((16, D), jnp.float32),
            pltpu.VMEM((CH, D), jnp.float32),
            pltpu.VMEM_SHARED((N8, D), jnp.float32),
        ],
    )
    return k(dstR)


def _sc_edge_body(*refs):
    (hw_hbm, gix_hbm, dst_hbm, agg_hbm,
     gv, dv, rowsA, rowsB, zb, aggS, semA, semB) = refs

    cid = lax.axis_index("c")
    sid = lax.axis_index("s")
    wid = sid * NC + cid
    base = sid * ROWS_PER_TILE

    zero16 = jnp.zeros((16,), jnp.float32)

    def _zb(i, _):
        zb[i // 8, pl.ds((i % 8) * 16, 16)] = zero16
        return 0
    lax.fori_loop(0, 128, _zb, 0)

    def _za(k, _):
        pltpu.sync_copy(zb, aggS.at[pl.ds(base + k * 16, 16)])
        return 0
    lax.fori_loop(0, ROWS_PER_TILE // 16, _za, 0)

    plsc.subcore_barrier()

    def _group(g, _):
        pltpu.sync_copy(gix_hbm.at[wid, pl.ds(g * GRP, GRP)], gv)
        pltpu.sync_copy(dst_hbm.at[wid, pl.ds(g * GRP, GRP)], dv)
        pltpu.async_copy(hw_hbm.at[gv.at[0]], rowsA, semA)

        def _pair(p, _):
            j0 = 2 * p
            j1 = 2 * p + 1
            pltpu.make_async_copy(hw_hbm.at[gv.at[j0]], rowsA, semA).wait()
            pltpu.async_copy(hw_hbm.at[gv.at[j1]], rowsB, semB)
            pltpu.sync_copy(rowsA, aggS.at[dv.at[j0]], add=True)
            pltpu.make_async_copy(hw_hbm.at[gv.at[j1]], rowsB, semB).wait()

            @pl.when(p < GRP // 2 - 1)
            def _prefetch():
                pltpu.async_copy(hw_hbm.at[gv.at[j1 + 1]], rowsA, semA)

            pltpu.sync_copy(rowsB, aggS.at[dv.at[j1]], add=True)
            return 0
        lax.fori_loop(0, GRP // 2, _pair, 0)
        return 0
    lax.fori_loop(0, NCHUNK // GRP, _group, 0)

    plsc.subcore_barrier()

    pltpu.sync_copy(aggS.at[pl.ds(base, ROWS_PER_TILE)],
                    agg_hbm.at[cid, pl.ds(base, ROWS_PER_TILE)])


def _sc_edge_pass(hW, gixR, dstR):
    mesh = plsc.VectorSubcoreMesh(core_axis_name="c", subcore_axis_name="s")
    k = pl.kernel(
        _sc_edge_body,
        out_type=jax.ShapeDtypeStruct((NC, N8, D), jnp.float32),
        mesh=mesh,
        scratch_types=[
            pltpu.VMEM((GRP, CH), jnp.int32),
            pltpu.VMEM((GRP, CH), jnp.int32),
            pltpu.VMEM((CH, D), jnp.float32),
            pltpu.VMEM((CH, D), jnp.float32),
            pltpu.VMEM((16, D), jnp.float32),
            pltpu.VMEM_SHARED((N8, D), jnp.float32),
            pltpu.SemaphoreType.DMA,
            pltpu.SemaphoreType.DMA,
        ],
    )
    return k(hW, gixR, dstR)


def _combine_body(agg_ref, deg_ref, h_ref, w_ref, b_ref, out_ref):
    a = agg_ref[0] + agg_ref[1]
    d = deg_ref[0, :, 0] + deg_ref[1, :, 0]
    inv = 1.0 / jnp.maximum(d, 1.0)
    m = a * inv[:, None]
    s = jnp.dot(h_ref[...], w_ref[...], preferred_element_type=jnp.float32)
    out_ref[...] = jnp.maximum(m + s + b_ref[...], 0.0)


def _combine(agg, deg, h, W_self, b):
    return pl.pallas_call(
        _combine_body,
        grid=(NB,),
        in_specs=[
            pl.BlockSpec((NC, BN, D), lambda i: (0, i, 0)),
            pl.BlockSpec((NC, BN, D), lambda i: (0, i, 0)),
            pl.BlockSpec((BN, D), lambda i: (i, 0)),
            pl.BlockSpec((D, D), lambda i: (0, 0)),
            pl.BlockSpec((1, D), lambda i: (0, 0)),
        ],
        out_specs=pl.BlockSpec((BN, D), lambda i: (i, 0)),
        out_shape=jax.ShapeDtypeStruct((N, D), jnp.float32),
    )(agg, deg, h, W_self, b)


def _readout_body(agg_ref, deg_ref, h_ref, w_ref, b_ref, ids_ref,
                  wc1_ref, bc1_ref, wc2_ref, bc2_ref, out_ref,
                  pooled_s, cnt_s):
    i = pl.program_id(0)

    @pl.when(i == 0)
    def _init():
        pooled_s[...] = jnp.zeros((G, D), jnp.float32)
        cnt_s[...] = jnp.zeros((G, 128), jnp.float32)

    a = agg_ref[0] + agg_ref[1]
    d = deg_ref[0, :, 0] + deg_ref[1, :, 0]
    inv = 1.0 / jnp.maximum(d, 1.0)
    m = a * inv[:, None]
    s = jnp.dot(h_ref[...], w_ref[...], preferred_element_type=jnp.float32)
    h2 = jnp.maximum(m + s + b_ref[...], 0.0)

    ids = ids_ref[0, 0, :]
    gi = lax.broadcasted_iota(jnp.int32, (G, BN), 0)
    mask = (ids[None, :] == gi).astype(jnp.float32)
    pooled_s[...] += jnp.dot(mask, h2, preferred_element_type=jnp.float32,
                             precision=lax.Precision.HIGHEST)
    cnt_s[:, 0:1] += jnp.sum(mask, axis=1, keepdims=True)

    @pl.when(i == NB - 1)
    def _final():
        cnt = jnp.maximum(cnt_s[:, 0:1], 1.0)
        pooled = pooled_s[...] / cnt
        hid = jnp.maximum(
            jnp.dot(pooled, wc1_ref[...], preferred_element_type=jnp.float32)
            + bc1_ref[...], 0.0)
        logits = jnp.sum(hid * wc2_ref[...], axis=1, keepdims=True) + bc2_ref[...]
        out_ref[...] = logits


def _readout(agg, deg, h, W_self, b, ids3, Wc1, bc1, wc2row, bc2):
    return pl.pallas_call(
        _readout_body,
        grid=(NB,),
        in_specs=[
            pl.BlockSpec((NC, BN, D), lambda i: (0, i, 0)),
            pl.BlockSpec((NC, BN, D), lambda i: (0, i, 0)),
            pl.BlockSpec((BN, D), lambda i: (i, 0)),
            pl.BlockSpec((D, D), lambda i: (0, 0)),
            pl.BlockSpec((1, D), lambda i: (0, 0)),
            pl.BlockSpec((1, 1, BN), lambda i: (i, 0, 0)),
            pl.BlockSpec((D, D), lambda i: (0, 0)),
            pl.BlockSpec((1, D), lambda i: (0, 0)),
            pl.BlockSpec((1, D), lambda i: (0, 0)),
            pl.BlockSpec((1, 1), lambda i: (0, 0)),
        ],
        out_specs=pl.BlockSpec((G, 1), lambda i: (0, 0)),
        out_shape=jax.ShapeDtypeStruct((G, 1), jnp.float32),
        scratch_shapes=[
            pltpu.VMEM((G, D), jnp.float32),
            pltpu.VMEM((G, 128), jnp.float32),
        ],
    )(agg, deg, h, W_self, b, ids3, Wc1, bc1, wc2row, bc2)


def kernel(feat, edge_index, edge_type, node_graph_ids,
           W_rel1, W_self1, b1, W_rel2, W_self2, b2,
           Wc1, bc1, Wc2, bc2):
    src = edge_index[0].astype(jnp.int32)
    dst = edge_index[1].astype(jnp.int32)
    ety = edge_type.astype(jnp.int32)

    pad = EP - E
    srcF = jnp.concatenate([src, jnp.zeros((pad,), jnp.int32)])
    etyF = jnp.concatenate([ety, jnp.zeros((pad,), jnp.int32)])
    dstR = jnp.concatenate([dst, jnp.full((pad,), N, jnp.int32)]
                           ).reshape(NW, NCHUNK, CH)
    gixR = _gidx_compute(srcF, etyF).reshape(NW, NCHUNK, CH)
    ids3 = node_graph_ids.astype(jnp.int32).reshape(NB, 1, BN)

    b1r = b1.reshape(1, D)
    b2r = b2.reshape(1, D)
    bc1r = bc1.reshape(1, D)
    wc2row = Wc2.reshape(1, D)
    bc2r = bc2.reshape(1, 1)

    deg = _sc_deg_pass(dstR)
    hW1 = _rel_matmul(feat, W_rel1)
    agg1 = _sc_edge_pass(hW1, gixR, dstR)
    h1 = _combine(agg1, deg, feat, W_self1, b1r)

    hW2 = _rel_matmul(h1, W_rel2)
    agg2 = _sc_edge_pass(hW2, gixR, dstR)
    logits = _readout(agg2, deg, h1, W_self2, b2r, ids3, Wc1, bc1r,
                      wc2row, bc2r)
    return logits
